# Initial kernel scaffold; baseline (speedup 1.0000x reference)
import jax
import jax.numpy as jnp
from jax.experimental import pallas as pl


def kernel(x, Wr, W1, b1, W2, b2):
    raise NotImplementedError("write your pallas kernel here")


# speedup vs baseline: 1.8932x; 1.8932x over previous
import functools

import jax
import jax.numpy as jnp
from jax import lax
from jax.experimental import pallas as pl
from jax.experimental.pallas import tpu as pltpu
from jax.experimental.pallas import tpu_sc as plsc

S = 2048
D = 768
H = 768
E = 64
K = 2
C = 96
NW = 32
JPW = (K * S) // NW
CH = 64
NCH = JPW // CH
PAD_ROWS = 8
EC = E * C


def _router_body(x_ref, wr_ref, g1_ref, g2_ref, d0_ref, d1_ref,
                 s0_ref, s1_ref, m0_ref, m1_ref, aux_ref):
    x = x_ref[...]
    wr = wr_ref[...]
    logits = jnp.dot(x, wr, preferred_element_type=jnp.float32)
    mx = jnp.max(logits, axis=1, keepdims=True)
    ex = jnp.exp(logits - mx)
    gates = ex / jnp.sum(ex, axis=1, keepdims=True)

    lane = lax.broadcasted_iota(jnp.int32, (S, E), 1)
    g1 = jnp.max(gates, axis=1, keepdims=True)
    i1 = jnp.min(jnp.where(gates == g1, lane, E), axis=1, keepdims=True)
    gates2 = jnp.where(lane == i1, -1.0, gates)
    g2 = jnp.max(gates2, axis=1, keepdims=True)
    i2 = jnp.min(jnp.where(gates2 == g2, lane, E), axis=1, keepdims=True)

    oh0 = (lane == i1).astype(jnp.float32)
    oh1 = (lane == i2).astype(jnp.float32)

    r = lax.broadcasted_iota(jnp.int32, (S, S), 0)
    c = lax.broadcasted_iota(jnp.int32, (S, S), 1)
    tri = (r >= c).astype(jnp.float32)
    cum0 = jnp.dot(tri, oh0, preferred_element_type=jnp.float32)
    tot0 = jnp.sum(oh0, axis=0, keepdims=True)
    cum1 = jnp.dot(tri, oh1, preferred_element_type=jnp.float32) + tot0

    pos0 = jnp.sum(cum0 * oh0, axis=1, keepdims=True) - 1.0
    pos1 = jnp.sum(cum1 * oh1, axis=1, keepdims=True) - 1.0
    w0 = (pos0 < C).astype(jnp.float32)
    w1 = (pos1 < C).astype(jnp.float32)
    p0 = jnp.clip(pos0, 0.0, C - 1.0).astype(jnp.int32)
    p1 = jnp.clip(pos1, 0.0, C - 1.0).astype(jnp.int32)
    d0 = i1 * C + p0
    d1 = i2 * C + p1

    srow = lax.broadcasted_iota(jnp.int32, (S, 1), 0)
    dump = EC + lax.rem(srow // JPW, PAD_ROWS)
    g1_ref[...] = g1
    g2_ref[...] = g2
    m0_ref[...] = w0
    m1_ref[...] = w1
    d0_ref[...] = jnp.where(w0 > 0.0, d0, 0)
    d1_ref[...] = jnp.where(w1 > 0.0, d1, 0)
    s0_ref[...] = jnp.where(w0 > 0.0, d0, dump)
    s1_ref[...] = jnp.where(w1 > 0.0, d1, dump)

    imp = jnp.sum(gates, axis=0, keepdims=True) * (1.0 / S)
    load = (jnp.sum(oh0 * w0, axis=0, keepdims=True)
            + jnp.sum(oh1 * w1, axis=0, keepdims=True)) * (1.0 / (K * S))
    aux_ref[...] = (E * jnp.sum(imp * load)).reshape(1, 1)


def _router(x2, wr):
    fl = jax.ShapeDtypeStruct((S, 1), jnp.float32)
    it = jax.ShapeDtypeStruct((S, 1), jnp.int32)
    return pl.pallas_call(
        _router_body,
        out_shape=(fl, fl, it, it, it, it, fl, fl,
                   jax.ShapeDtypeStruct((1, 1), jnp.float32)),
    )(x2, wr)


def _dispatch_body(x_hbm, ds_hbm, out_hbm, idx_v, rows_v, sem):
    wid = lax.axis_index("s") * 2 + lax.axis_index("c")
    src0 = lax.rem(wid, 16) * JPW
    pltpu.sync_copy(ds_hbm.at[wid], idx_v)
    for ch in range(NCH):
        pltpu.sync_copy(x_hbm.at[pl.ds(src0 + ch * CH, CH)], rows_v)
        pltpu.async_copy(rows_v, out_hbm.at[idx_v.at[ch]], sem).wait()


def _dispatch(x2, ds3):
    mesh = plsc.VectorSubcoreMesh(core_axis_name="c", subcore_axis_name="s")
    f = functools.partial(
        pl.kernel, _dispatch_body, mesh=mesh,
        out_type=jax.ShapeDtypeStruct((EC + PAD_ROWS, D), jnp.float32),
        scratch_types=[
            pltpu.VMEM((NCH, CH), jnp.int32),
            pltpu.VMEM((CH, D), jnp.float32),
            pltpu.SemaphoreType.DMA,
        ],
    )()
    return f(x2, ds3)


def _ffn_body(in_ref, w1_ref, b1_ref, w2_ref, b2_ref, out_ref):
    h = jnp.dot(in_ref[...], w1_ref[0], preferred_element_type=jnp.float32)
    h = jnp.maximum(h + b1_ref[0], 0.0)
    out_ref[...] = (jnp.dot(h, w2_ref[0], preferred_element_type=jnp.float32)
                    + b2_ref[0])


def _ffn(expert_in, w1, b1, w2, b2):
    return pl.pallas_call(
        _ffn_body,
        grid=(E,),
        in_specs=[
            pl.BlockSpec((C, D), lambda e: (e, 0)),
            pl.BlockSpec((1, D, H), lambda e: (e, 0, 0)),
            pl.BlockSpec((1, 1, H), lambda e: (e, 0, 0)),
            pl.BlockSpec((1, H, D), lambda e: (e, 0, 0)),
            pl.BlockSpec((1, 1, D), lambda e: (e, 0, 0)),
        ],
        out_specs=pl.BlockSpec((C, D), lambda e: (e, 0)),
        out_shape=jax.ShapeDtypeStruct((EC, D), jnp.float32),
    )(expert_in, w1, b1.reshape(E, 1, H), w2, b2.reshape(E, 1, D))


def _gather_body(eo_hbm, dd_hbm, out_hbm, idx_v, rows_v, sem):
    wid = lax.axis_index("s") * 2 + lax.axis_index("c")
    pltpu.sync_copy(dd_hbm.at[wid], idx_v)
    for ch in range(NCH):
        pltpu.async_copy(eo_hbm.at[idx_v.at[ch]], rows_v, sem).wait()
        pltpu.sync_copy(rows_v, out_hbm.at[pl.ds(wid * JPW + ch * CH, CH)])


def _gather(eo, dd3):
    mesh = plsc.VectorSubcoreMesh(core_axis_name="c", subcore_axis_name="s")
    f = functools.partial(
        pl.kernel, _gather_body, mesh=mesh,
        out_type=jax.ShapeDtypeStruct((K * S, D), jnp.float32),
        scratch_types=[
            pltpu.VMEM((NCH, CH), jnp.int32),
            pltpu.VMEM((CH, D), jnp.float32),
            pltpu.SemaphoreType.DMA,
        ],
    )()
    return f(eo, dd3)


def _combine_body(r0_ref, r1_ref, g1_ref, g2_ref, m0_ref, m1_ref, out_ref):
    z = jnp.zeros_like(r0_ref[...])
    c0 = jnp.where(jnp.broadcast_to(m0_ref[...] > 0.0, z.shape),
                   g1_ref[...] * r0_ref[...], z)
    c1 = jnp.where(jnp.broadcast_to(m1_ref[...] > 0.0, z.shape),
                   g2_ref[...] * r1_ref[...], z)
    out_ref[...] = c0 + c1


def _combine(gathered, g1, g2, m0, m1):
    nb = 8
    rb = S // nb
    col = pl.BlockSpec((rb, 1), lambda i: (i, 0))
    return pl.pallas_call(
        _combine_body,
        grid=(nb,),
        in_specs=[
            pl.BlockSpec((rb, D), lambda i: (i, 0)),
            pl.BlockSpec((rb, D), lambda i: (i + nb, 0)),
            col, col, col, col,
        ],
        out_specs=pl.BlockSpec((rb, D), lambda i: (i, 0)),
        out_shape=jax.ShapeDtypeStruct((S, D), jnp.float32),
    )(gathered, gathered, g1, g2, m0, m1)


def kernel(x, Wr, W1, b1, W2, b2):
    x2 = x.reshape(S, D)
    g1, g2, d0, d1, s0, s1, m0, m1, aux = _router(x2, Wr)
    ds3 = jnp.concatenate([s0, s1], axis=1).T.reshape(NW, NCH, CH)
    dd3 = jnp.concatenate([d0, d1], axis=1).T.reshape(NW, NCH, CH)
    expert_in = _dispatch(x2, ds3)
    eo = _ffn(expert_in, W1, b1, W2, b2)
    gathered = _gather(eo, dd3)
    out = _combine(gathered, g1, g2, m0, m1)
    return out.reshape(1, S, D), aux[0, 0]
